# baseline (speedup 1.0000x reference)
import functools

import jax
import jax.numpy as jnp
from jax import lax
from jax.experimental import pallas as pl
from jax.experimental.pallas import tpu as pltpu
from jax.experimental.pallas import tpu_sc as plsc

N = 10000
D = 128
R = 8
E = 320000

NB = 1000
NBLK = N // NB

NC = 2
NS = 16
NW = NC * NS
CHUNK = 32
CPW = 320
EPW = CPW * CHUNK
E_PAD = EPW * NW
DUMP = N
RPT = 632
N_ACC = RPT * NS
NBUF = 8
WIN = 40


AIB = NW * CPW // (NBLK * R)


def _xscaled_body(emb_ref, wr_ref, remb_ref, src_ref, et_ref, out_ref, aidx_ref):
    emb = emb_ref[...]
    x = jnp.dot(emb, wr_ref[0], preferred_element_type=jnp.float32)
    att = jnp.sum(x * jnp.tanh(x + remb_ref[0]), axis=1, keepdims=True)
    out_ref[...] = emb * att
    aidx_ref[...] = et_ref[...] * N + src_ref[...]


def _make_xscaled(emb, W_R, rel_embed, src, et):
    return pl.pallas_call(
        _xscaled_body,
        grid=(NBLK, R),
        in_specs=[
            pl.BlockSpec((NB, D), lambda i, r: (i, 0)),
            pl.BlockSpec((1, D, D), lambda i, r: (r, 0, 0)),
            pl.BlockSpec((1, 1, D), lambda i, r: (r, 0, 0)),
            pl.BlockSpec((AIB, CHUNK), lambda i, r: (i * R + r, 0)),
            pl.BlockSpec((AIB, CHUNK), lambda i, r: (i * R + r, 0)),
        ],
        out_specs=[
            pl.BlockSpec((NB, D), lambda i, r: (r * NBLK + i, 0)),
            pl.BlockSpec((AIB, CHUNK), lambda i, r: (i * R + r, 0)),
        ],
        out_shape=[
            jax.ShapeDtypeStruct((R * N, D), jnp.float32),
            jax.ShapeDtypeStruct((NW * CPW, CHUNK), jnp.int32),
        ],
    )(emb, W_R, rel_embed.reshape(R, 1, D), src, et)


_SC_MESH = plsc.VectorSubcoreMesh(core_axis_name="c", subcore_axis_name="s")


_EXPORT_SIZES = (32,) * 19 + (24,)


@functools.partial(
    pl.kernel,
    mesh=_SC_MESH,
    out_type=jax.ShapeDtypeStruct((NC * N_ACC, D), jnp.float32),
    scratch_types=[
        pltpu.VMEM((WIN, CHUNK), jnp.int32),
        pltpu.VMEM((WIN, CHUNK), jnp.int32),
        pltpu.VMEM((CHUNK, D), jnp.float32),
        pltpu.VMEM((CHUNK, D), jnp.float32),
        pltpu.VMEM((CHUNK, D), jnp.float32),
        pltpu.VMEM((CHUNK, D), jnp.float32),
        pltpu.VMEM((CHUNK, D), jnp.float32),
        pltpu.VMEM((CHUNK, D), jnp.float32),
        pltpu.VMEM((CHUNK, D), jnp.float32),
        pltpu.VMEM((CHUNK, D), jnp.float32),
        pltpu.VMEM_SHARED((N_ACC, D), jnp.float32),
        pltpu.SemaphoreType.DMA,
        pltpu.SemaphoreType.DMA,
        pltpu.SemaphoreType.DMA,
        pltpu.SemaphoreType.DMA,
        pltpu.SemaphoreType.DMA,
        pltpu.SemaphoreType.DMA,
        pltpu.SemaphoreType.DMA,
        pltpu.SemaphoreType.DMA,
        pltpu.SemaphoreType.DMA,
        pltpu.SemaphoreType.DMA,
        pltpu.SemaphoreType.DMA,
        pltpu.SemaphoreType.DMA,
        pltpu.SemaphoreType.DMA,
        pltpu.SemaphoreType.DMA,
        pltpu.SemaphoreType.DMA,
        pltpu.SemaphoreType.DMA,
    ],
)
def _sc_aggregate(xs_hbm, aidx_hbm, dst_hbm, out_hbm,
                  aidx, dbuf, r0, r1, r2, r3, r4, r5, r6, r7, acc,
                  g0, g1, g2, g3, g4, g5, g6, g7,
                  s0, s1, s2, s3, s4, s5, s6, s7):
    rows = (r0, r1, r2, r3, r4, r5, r6, r7)
    gsem = (g0, g1, g2, g3, g4, g5, g6, g7)
    ssem = (s0, s1, s2, s3, s4, s5, s6, s7)
    c = lax.axis_index("c")
    s = lax.axis_index("s")
    wid = s * NC + c

    zv = jnp.zeros((16,), jnp.float32)

    def _zrow(i, carry):
        for j in range(D // 16):
            r0[i, pl.ds(j * 16, 16)] = zv
        return carry

    lax.fori_loop(0, CHUNK, _zrow, 0)
    off = 0
    for sz in _EXPORT_SIZES:
        pltpu.sync_copy(r0.at[pl.ds(0, sz)], acc.at[pl.ds(s * RPT + off, sz)])
        off += sz

    plsc.subcore_barrier()

    for w in range(CPW // WIN):
        base = wid * CPW + w * WIN
        pltpu.sync_copy(aidx_hbm.at[pl.ds(base, WIN)], aidx)
        pltpu.sync_copy(dst_hbm.at[pl.ds(base, WIN)], dbuf)

        for b in range(NBUF):
            pltpu.async_copy(xs_hbm.at[aidx.at[b]], rows[b], gsem[b])

        def _group(g, carry):
            for b in range(NBUF):
                ci = g * NBUF + b
                pltpu.make_async_copy(xs_hbm.at[aidx.at[ci]], rows[b], gsem[b]).wait()
                pltpu.async_copy(rows[b], acc.at[dbuf.at[ci]], ssem[b], add=True)

            for b in range(NBUF):
                ci = g * NBUF + b

                @pl.when(g < WIN // NBUF - 1)
                def _rearm():
                    pltpu.make_async_copy(rows[b], acc.at[dbuf.at[ci]], ssem[b]).wait()
                    pltpu.async_copy(xs_hbm.at[aidx.at[ci + NBUF]], rows[b], gsem[b])

                @pl.when(g == WIN // NBUF - 1)
                def _drain():
                    pltpu.make_async_copy(rows[b], acc.at[dbuf.at[ci]], ssem[b]).wait()
            return carry

        lax.fori_loop(0, WIN // NBUF, _group, 0)

    plsc.subcore_barrier()

    off = 0
    for sz in _EXPORT_SIZES:
        r_lo = s * RPT + off
        pltpu.sync_copy(acc.at[pl.ds(r_lo, sz)], r0.at[pl.ds(0, sz)])
        pltpu.sync_copy(r0.at[pl.ds(0, sz)], out_hbm.at[pl.ds(c * N_ACC + r_lo, sz)])
        off += sz


def _out_body(emb_ref, p0_ref, p1_ref, w1_ref, w2_ref, b_ref, out_ref):
    emb = emb_ref[...]
    nf = p0_ref[...] + p1_ref[...]
    h = (jnp.dot(emb, w1_ref[...], preferred_element_type=jnp.float32)
         + jnp.dot(nf, w2_ref[...], preferred_element_type=jnp.float32)
         + b_ref[...])
    out_ref[...] = jnp.where(h >= 0, h, 0.01 * h) + emb


def _make_out(emb, p0, p1, w1t, w2t, b):
    return pl.pallas_call(
        _out_body,
        grid=(NBLK,),
        in_specs=[
            pl.BlockSpec((NB, D), lambda i: (i, 0)),
            pl.BlockSpec((NB, D), lambda i: (i, 0)),
            pl.BlockSpec((NB, D), lambda i: (i, 0)),
            pl.BlockSpec((D, D), lambda i: (0, 0)),
            pl.BlockSpec((D, D), lambda i: (0, 0)),
            pl.BlockSpec((1, D), lambda i: (0, 0)),
        ],
        out_specs=pl.BlockSpec((NB, D), lambda i: (i, 0)),
        out_shape=jax.ShapeDtypeStruct((N, D), jnp.float32),
    )(emb, p0, p1, w1t, w2t, b)


def kernel(entity_embed, edge_index, edge_type, rel_embed, W_R, W_w, W_b):
    emb = entity_embed.astype(jnp.float32)
    src = edge_index[0].astype(jnp.int32)
    dst = edge_index[1].astype(jnp.int32)
    et = edge_type.astype(jnp.int32)

    pad = E_PAD - E
    pad_src = jax.lax.iota(jnp.int32, pad) % N
    pad_dst = DUMP + (jax.lax.iota(jnp.int32, pad) % (N_ACC - N))
    src = jnp.concatenate([src, pad_src]).reshape(NW * CPW, CHUNK)
    dst = jnp.concatenate([dst, pad_dst]).reshape(NW * CPW, CHUNK)
    et = jnp.concatenate([et, jnp.zeros((pad,), jnp.int32)]).reshape(NW * CPW, CHUNK)

    xs, aidx = _make_xscaled(
        emb, W_R.astype(jnp.float32), rel_embed.astype(jnp.float32), src, et)
    partials = _sc_aggregate(xs, aidx, dst)
    p0 = partials[:N]
    p1 = partials[N_ACC:N_ACC + N]

    w1t = W_w[:, :D].T
    w2t = W_w[:, D:].T
    return _make_out(emb, p0, p1, w1t, w2t, W_b.reshape(1, D))


# speedup vs baseline: 1.0557x; 1.0557x over previous
import functools

import jax
import jax.numpy as jnp
from jax import lax
from jax.experimental import pallas as pl
from jax.experimental.pallas import tpu as pltpu
from jax.experimental.pallas import tpu_sc as plsc

N = 10000
D = 128
R = 8
E = 320000

NB = 1000
NBLK = N // NB

NC = 2
NS = 16
NW = NC * NS
CHUNK = 64
CPW = 160
EPW = CPW * CHUNK
E_PAD = EPW * NW
DUMP = N
RPT = 632
N_ACC = RPT * NS
NBUF = 4
WIN = 40


AIB = NW * CPW // (NBLK * R)


def _xscaled_body(emb_ref, wr_ref, remb_ref, src_ref, et_ref, out_ref, aidx_ref):
    emb = emb_ref[...]
    x = jnp.dot(emb, wr_ref[0], preferred_element_type=jnp.float32)
    att = jnp.sum(x * jnp.tanh(x + remb_ref[0]), axis=1, keepdims=True)
    out_ref[...] = emb * att
    aidx_ref[...] = et_ref[...] * N + src_ref[...]


def _make_xscaled(emb, W_R, rel_embed, src, et):
    return pl.pallas_call(
        _xscaled_body,
        grid=(NBLK, R),
        in_specs=[
            pl.BlockSpec((NB, D), lambda i, r: (i, 0)),
            pl.BlockSpec((1, D, D), lambda i, r: (r, 0, 0)),
            pl.BlockSpec((1, 1, D), lambda i, r: (r, 0, 0)),
            pl.BlockSpec((AIB, CHUNK), lambda i, r: (i * R + r, 0)),
            pl.BlockSpec((AIB, CHUNK), lambda i, r: (i * R + r, 0)),
        ],
        out_specs=[
            pl.BlockSpec((NB, D), lambda i, r: (r * NBLK + i, 0)),
            pl.BlockSpec((AIB, CHUNK), lambda i, r: (i * R + r, 0)),
        ],
        out_shape=[
            jax.ShapeDtypeStruct((R * N, D), jnp.float32),
            jax.ShapeDtypeStruct((NW * CPW, CHUNK), jnp.int32),
        ],
    )(emb, W_R, rel_embed.reshape(R, 1, D), src, et)


_SC_MESH = plsc.VectorSubcoreMesh(core_axis_name="c", subcore_axis_name="s")


_EXPORT_SIZES = (64,) * 9 + (56,)


@functools.partial(
    pl.kernel,
    mesh=_SC_MESH,
    out_type=jax.ShapeDtypeStruct((NC * N_ACC, D), jnp.float32),
    scratch_types=[
        pltpu.VMEM((WIN, CHUNK), jnp.int32),
        pltpu.VMEM((WIN, CHUNK), jnp.int32),
        pltpu.VMEM((CHUNK, D), jnp.float32),
        pltpu.VMEM((CHUNK, D), jnp.float32),
        pltpu.VMEM((CHUNK, D), jnp.float32),
        pltpu.VMEM((CHUNK, D), jnp.float32),
        pltpu.VMEM_SHARED((N_ACC, D), jnp.float32),
        pltpu.SemaphoreType.DMA,
        pltpu.SemaphoreType.DMA,
        pltpu.SemaphoreType.DMA,
        pltpu.SemaphoreType.DMA,
        pltpu.SemaphoreType.DMA,
        pltpu.SemaphoreType.DMA,
        pltpu.SemaphoreType.DMA,
        pltpu.SemaphoreType.DMA,
    ],
)
def _sc_aggregate(xs_hbm, aidx_hbm, dst_hbm, out_hbm,
                  aidx, dbuf, r0, r1, r2, r3, acc,
                  g0, g1, g2, g3, s0, s1, s2, s3):
    rows = (r0, r1, r2, r3)
    gsem = (g0, g1, g2, g3)
    ssem = (s0, s1, s2, s3)
    c = lax.axis_index("c")
    s = lax.axis_index("s")
    wid = s * NC + c

    zv = jnp.zeros((16,), jnp.float32)

    def _zrow(i, carry):
        for j in range(D // 16):
            r0[i, pl.ds(j * 16, 16)] = zv
        return carry

    lax.fori_loop(0, CHUNK, _zrow, 0)
    off = 0
    for sz in _EXPORT_SIZES:
        pltpu.sync_copy(r0.at[pl.ds(0, sz)], acc.at[pl.ds(s * RPT + off, sz)])
        off += sz

    plsc.subcore_barrier()

    for w in range(CPW // WIN):
        base = wid * CPW + w * WIN
        pltpu.sync_copy(aidx_hbm.at[pl.ds(base, WIN)], aidx)
        pltpu.sync_copy(dst_hbm.at[pl.ds(base, WIN)], dbuf)

        for b in range(NBUF):
            pltpu.async_copy(xs_hbm.at[aidx.at[b]], rows[b], gsem[b])

        def _group(g, carry):
            for b in range(NBUF):
                ci = g * NBUF + b
                pltpu.make_async_copy(xs_hbm.at[aidx.at[ci]], rows[b], gsem[b]).wait()
                pltpu.async_copy(rows[b], acc.at[dbuf.at[ci]], ssem[b], add=True)

            for b in range(NBUF):
                ci = g * NBUF + b

                @pl.when(g < WIN // NBUF - 1)
                def _rearm():
                    pltpu.make_async_copy(rows[b], acc.at[dbuf.at[ci]], ssem[b]).wait()
                    pltpu.async_copy(xs_hbm.at[aidx.at[ci + NBUF]], rows[b], gsem[b])

                @pl.when(g == WIN // NBUF - 1)
                def _drain():
                    pltpu.make_async_copy(rows[b], acc.at[dbuf.at[ci]], ssem[b]).wait()
            return carry

        lax.fori_loop(0, WIN // NBUF, _group, 0)

    plsc.subcore_barrier()

    off = 0
    for sz in _EXPORT_SIZES:
        r_lo = s * RPT + off
        pltpu.sync_copy(acc.at[pl.ds(r_lo, sz)], r0.at[pl.ds(0, sz)])
        pltpu.sync_copy(r0.at[pl.ds(0, sz)], out_hbm.at[pl.ds(c * N_ACC + r_lo, sz)])
        off += sz


def _out_body(emb_ref, p0_ref, p1_ref, w1_ref, w2_ref, b_ref, out_ref):
    emb = emb_ref[...]
    nf = p0_ref[...] + p1_ref[...]
    h = (jnp.dot(emb, w1_ref[...], preferred_element_type=jnp.float32)
         + jnp.dot(nf, w2_ref[...], preferred_element_type=jnp.float32)
         + b_ref[...])
    out_ref[...] = jnp.where(h >= 0, h, 0.01 * h) + emb


def _make_out(emb, p0, p1, w1t, w2t, b):
    return pl.pallas_call(
        _out_body,
        grid=(NBLK,),
        in_specs=[
            pl.BlockSpec((NB, D), lambda i: (i, 0)),
            pl.BlockSpec((NB, D), lambda i: (i, 0)),
            pl.BlockSpec((NB, D), lambda i: (i, 0)),
            pl.BlockSpec((D, D), lambda i: (0, 0)),
            pl.BlockSpec((D, D), lambda i: (0, 0)),
            pl.BlockSpec((1, D), lambda i: (0, 0)),
        ],
        out_specs=pl.BlockSpec((NB, D), lambda i: (i, 0)),
        out_shape=jax.ShapeDtypeStruct((N, D), jnp.float32),
    )(emb, p0, p1, w1t, w2t, b)


def kernel(entity_embed, edge_index, edge_type, rel_embed, W_R, W_w, W_b):
    emb = entity_embed.astype(jnp.float32)
    src = edge_index[0].astype(jnp.int32)
    dst = edge_index[1].astype(jnp.int32)
    et = edge_type.astype(jnp.int32)

    pad = E_PAD - E
    pad_src = jax.lax.iota(jnp.int32, pad) % N
    pad_dst = DUMP + (jax.lax.iota(jnp.int32, pad) % (N_ACC - N))
    src = jnp.concatenate([src, pad_src]).reshape(NW * CPW, CHUNK)
    dst = jnp.concatenate([dst, pad_dst]).reshape(NW * CPW, CHUNK)
    et = jnp.concatenate([et, jnp.zeros((pad,), jnp.int32)]).reshape(NW * CPW, CHUNK)

    xs, aidx = _make_xscaled(
        emb, W_R.astype(jnp.float32), rel_embed.astype(jnp.float32), src, et)
    partials = _sc_aggregate(xs, aidx, dst)
    p0 = partials[:N]
    p1 = partials[N_ACC:N_ACC + N]

    w1t = W_w[:, :D].T
    w2t = W_w[:, D:].T
    return _make_out(emb, p0, p1, w1t, w2t, W_b.reshape(1, D))
